# baseline (speedup 1.0000x reference)
import functools

import jax
import jax.numpy as jnp
from jax import lax
from jax.experimental import pallas as pl
from jax.experimental.pallas import tpu as pltpu, tpu_sc as plsc

MAX_NUM_TILES = 4
NUM_PATCHES = 1025
HIDDEN = 1280
BATCH = 8

D = NUM_PATCHES * HIDDEN
ROWS = BATCH * MAX_NUM_TILES
TROWS = 9 * MAX_NUM_TILES
NSUB = 32
COLS = D // NSUB
NCH = 5
CH = COLS // NCH
NV = CH // 16 + 1
NCHUNKS = ROWS * NCH


def _body(hid, tbl, emb, rid, gat, out,
          emb_v, rid_v, gat_v,
          h0, h1, t0, t1, o0, o1,
          si0, si1, so0, so1):
    wid = lax.axis_index("s") * 2 + lax.axis_index("c")
    base = wid * COLS

    pltpu.sync_copy(rid, rid_v)
    pltpu.sync_copy(gat, gat_v)
    pltpu.sync_copy(emb.at[pl.ds(base, COLS)], emb_v)

    ridv = rid_v[...]
    graw = gat_v[...]
    ax = jnp.abs(graw)
    ex = jnp.exp(ax * 2.0)
    tpos = 1.0 - 2.0 / (ex + 1.0)
    gv = jnp.where(graw < 0.0, -tpos, tpos)
    omgv = 1.0 - gv
    lanes = lax.iota(jnp.int32, 16)

    hbuf = (h0, h1)
    tbuf = (t0, t1)
    obuf = (o0, o1)
    sin = (si0, si1)
    sout = (so0, so1)

    def chunk_idx(k):
        r = k // NCH
        c = k - r * NCH
        b = r // MAX_NUM_TILES
        t = r - b * MAX_NUM_TILES
        tr = jnp.sum(jnp.where(lanes == b, ridv, 0)) + t
        col = base + c * CH
        return r, c, tr, col

    def start_in(k, s):
        r, _, tr, col = chunk_idx(k)
        pltpu.async_copy(hid.at[r, pl.ds(col, CH)], hbuf[s], sin[s])
        pltpu.async_copy(tbl.at[tr, pl.ds(col, CH)], tbuf[s], sin[s])

    def wait_in(s):
        pltpu.make_async_copy(hid.at[0, pl.ds(0, CH)], hbuf[s], sin[s]).wait()
        pltpu.make_async_copy(hid.at[0, pl.ds(0, CH)], tbuf[s], sin[s]).wait()

    def start_out(k, s):
        r, _, _, col = chunk_idx(k)
        pltpu.async_copy(obuf[s], out.at[r, pl.ds(col, CH)], sout[s])

    def wait_out(s):
        pltpu.make_async_copy(obuf[s], out.at[0, pl.ds(0, CH)], sout[s]).wait()

    def compute(k, s):
        c = k - (k // NCH) * NCH
        eoff = c * CH
        hs, ts, os = hbuf[s], tbuf[s], obuf[s]

        def vstep(i, carry):
            off = jnp.minimum(i * 16, CH - 16)
            e = emb_v[pl.ds(eoff + off, 16)]
            h = hs[pl.ds(off, 16)]
            t = ts[pl.ds(off, 16)]
            os[pl.ds(off, 16)] = h + omgv * e + gv * t
            return carry

        lax.fori_loop(0, NV, vstep, 0)

    start_in(0, 0)
    start_in(1, 1)

    def outer(j, carry):
        for s in (0, 1):
            k = j * 2 + s
            wait_in(s)

            @pl.when(j >= 1)
            def _():
                wait_out(s)

            compute(k, s)
            start_out(k, s)

            @pl.when(k + 2 < NCHUNKS)
            def _():
                start_in(k + 2, s)
        return carry

    lax.fori_loop(0, NCHUNKS // 2, outer, 0)
    wait_out(0)
    wait_out(1)


@jax.jit
def kernel(hidden_state, aspect_ratio_ids, gate, embedding, tile_table):
    hid = hidden_state.reshape(ROWS, D)
    tbl = tile_table.reshape(TROWS, D)
    emb = embedding.reshape(D)
    rid = jnp.zeros((16,), jnp.int32).at[:BATCH].set(
        aspect_ratio_ids.astype(jnp.int32) * MAX_NUM_TILES)
    gat = jnp.broadcast_to(gate.astype(jnp.float32), (16,))

    run = pl.kernel(
        _body,
        out_type=jax.ShapeDtypeStruct((ROWS, D), jnp.float32),
        mesh=plsc.VectorSubcoreMesh(
            core_axis_name="c", subcore_axis_name="s",
            num_cores=2, num_subcores=16),
        compiler_params=pltpu.CompilerParams(
            use_tc_tiling_on_sc=False, needs_layout_passes=False),
        scratch_types=[
            pltpu.VMEM((COLS,), jnp.float32),
            pltpu.VMEM((16,), jnp.int32),
            pltpu.VMEM((16,), jnp.float32),
            pltpu.VMEM((CH,), jnp.float32),
            pltpu.VMEM((CH,), jnp.float32),
            pltpu.VMEM((CH,), jnp.float32),
            pltpu.VMEM((CH,), jnp.float32),
            pltpu.VMEM((CH,), jnp.float32),
            pltpu.VMEM((CH,), jnp.float32),
            pltpu.SemaphoreType.DMA,
            pltpu.SemaphoreType.DMA,
            pltpu.SemaphoreType.DMA,
            pltpu.SemaphoreType.DMA,
        ],
    )
    out = run(hid, tbl, emb, rid, gat)
    return out.reshape(BATCH, MAX_NUM_TILES, NUM_PATCHES, HIDDEN)


# speedup vs baseline: 7.0006x; 7.0006x over previous
import jax
import jax.numpy as jnp
from jax import lax
from jax.experimental import pallas as pl
from jax.experimental.pallas import tpu as pltpu, tpu_sc as plsc

MAX_NUM_TILES = 4
NUM_PATCHES = 1025
HIDDEN = 1280
BATCH = 8
NPH = NUM_PATCHES * HIDDEN
NSUB = 32


def _body(hid_t, tbl, tbl8, embb, rid, gat, out_t,
          tbuf, t8buf, ebuf, hb0, hb1, ob0, ob1,
          rid_v, gat_v,
          st, sh0, sh1, so0, so1):
    wid = lax.axis_index("s") * 2 + lax.axis_index("c")

    pltpu.sync_copy(rid, rid_v)
    pltpu.sync_copy(gat, gat_v)
    ridv = rid_v[...]
    graw = gat_v[...]
    ax = jnp.abs(graw)
    ex = jnp.exp(ax * 2.0)
    tpos = 1.0 - 2.0 / (ex + 1.0)
    gv = jnp.where(graw < 0.0, -tpos, tpos)
    omgv = 1.0 - gv
    lanes = lax.iota(jnp.int32, 16)
    any8 = jnp.max(ridv) == 8

    hb = (hb0, hb1)
    ob = (ob0, ob1)
    sh = (sh0, sh1)
    so = (so0, so1)

    n_k = jnp.where(wid == 0, 33, 32)

    def p_of(k):
        return wid + NSUB * k

    def start_tbl(k):
        p = p_of(k)
        c0 = p * HIDDEN
        for t in range(MAX_NUM_TILES):
            pltpu.async_copy(
                tbl.at[pl.ds(0, 8), pl.ds(t * NPH + c0, HIDDEN)],
                tbuf.at[t], st)
        pltpu.async_copy(embb.at[p], ebuf, st)

        @pl.when(any8)
        def _():
            pltpu.async_copy(tbl8.at[p], t8buf, st)

    def wait_tbl():
        for t in range(MAX_NUM_TILES):
            pltpu.make_async_copy(
                tbl.at[pl.ds(0, 8), pl.ds(0, HIDDEN)], tbuf.at[t],
                st).wait()
        pltpu.make_async_copy(embb.at[0], ebuf, st).wait()

        @pl.when(any8)
        def _():
            pltpu.make_async_copy(tbl8.at[0], t8buf, st).wait()

    def start_h(k, b, s):
        pltpu.async_copy(hid_t.at[b, p_of(k)], hb[s], sh[s])

    def wait_h(s):
        pltpu.make_async_copy(hid_t.at[0, 0], hb[s], sh[s]).wait()

    def start_o(k, b, s):
        pltpu.async_copy(ob[s], out_t.at[b, p_of(k)], so[s])

    def wait_o(s):
        pltpu.make_async_copy(ob[s], out_t.at[0, 0], so[s]).wait()

    def compute(b, s_h):
        idb = jnp.sum(jnp.where(lanes == b, ridv, 0))
        hbuf, obuf = hb[s_h], ob[s_h]

        @pl.when(idb < 8)
        def _():
            for t in range(MAX_NUM_TILES):
                def cbody(ci, carry, t=t):
                    for jj in range(4):
                        c = ci * 64 + jj * 16
                        obuf[t, pl.ds(c, 16)] = (
                            hbuf[t, pl.ds(c, 16)]
                            + omgv * ebuf[t, pl.ds(c, 16)]
                            + gv * tbuf[t, idb, pl.ds(c, 16)])
                    return carry
                lax.fori_loop(0, HIDDEN // 64, cbody, 0)

        @pl.when(idb == 8)
        def _():
            for t in range(MAX_NUM_TILES):
                def cbody(ci, carry, t=t):
                    for jj in range(4):
                        c = ci * 64 + jj * 16
                        obuf[t, pl.ds(c, 16)] = (
                            hbuf[t, pl.ds(c, 16)]
                            + omgv * ebuf[t, pl.ds(c, 16)]
                            + gv * t8buf[t, pl.ds(c, 16)])
                    return carry
                lax.fori_loop(0, HIDDEN // 64, cbody, 0)

    start_tbl(0)
    start_h(0, 0, 0)

    def chunk(k, carry):
        wait_tbl()

        def bpair(q, carry2, k=k):
            for sb in (0, 1):
                b = 2 * q + sb
                wait_h(sb)
                if sb == 0:
                    start_h(k, b + 1, 1)
                else:
                    @pl.when(q < 3)
                    def _(k=k, b=b):
                        start_h(k, b + 1, 0)

                    @pl.when((q == 3) & (k + 1 < n_k))
                    def _(k=k):
                        start_h(k + 1, 0, 0)
                @pl.when((q >= 1) | (k > 0))
                def _(sb=sb):
                    wait_o(sb)
                compute(b, sb)
                start_o(k, b, sb)
            return carry2

        lax.fori_loop(0, BATCH // 2, bpair, 0)

        @pl.when(k + 1 < n_k)
        def _(k=k):
            start_tbl(k + 1)
        return carry

    lax.fori_loop(0, n_k, chunk, 0)
    wait_o(0)
    wait_o(1)


@jax.jit
def kernel(hidden_state, aspect_ratio_ids, gate, embedding, tile_table):
    hid_t = jnp.transpose(hidden_state, (0, 2, 1, 3))
    tbl8 = jnp.transpose(
        tile_table[8].reshape(MAX_NUM_TILES, NUM_PATCHES, HIDDEN),
        (1, 0, 2))
    embb = jnp.broadcast_to(
        embedding[:, None, :],
        (NUM_PATCHES, MAX_NUM_TILES, HIDDEN))
    rid = jnp.zeros((16,), jnp.int32).at[:BATCH].set(
        aspect_ratio_ids.astype(jnp.int32))
    gat = jnp.broadcast_to(gate.astype(jnp.float32), (16,))

    run = pl.kernel(
        _body,
        out_type=jax.ShapeDtypeStruct(
            (BATCH, NUM_PATCHES, MAX_NUM_TILES, HIDDEN), jnp.float32),
        mesh=plsc.VectorSubcoreMesh(
            core_axis_name="c", subcore_axis_name="s",
            num_cores=2, num_subcores=16),
        compiler_params=pltpu.CompilerParams(needs_layout_passes=False),
        scratch_types=[
            pltpu.VMEM((MAX_NUM_TILES, 8, HIDDEN), jnp.float32),
            pltpu.VMEM((MAX_NUM_TILES, HIDDEN), jnp.float32),
            pltpu.VMEM((MAX_NUM_TILES, HIDDEN), jnp.float32),
            pltpu.VMEM((MAX_NUM_TILES, HIDDEN), jnp.float32),
            pltpu.VMEM((MAX_NUM_TILES, HIDDEN), jnp.float32),
            pltpu.VMEM((MAX_NUM_TILES, HIDDEN), jnp.float32),
            pltpu.VMEM((MAX_NUM_TILES, HIDDEN), jnp.float32),
            pltpu.VMEM((16,), jnp.int32),
            pltpu.VMEM((16,), jnp.float32),
            pltpu.SemaphoreType.DMA,
            pltpu.SemaphoreType.DMA,
            pltpu.SemaphoreType.DMA,
            pltpu.SemaphoreType.DMA,
            pltpu.SemaphoreType.DMA,
        ],
    )
    out_t = run(hid_t, tile_table, tbl8, embb, rid, gat)
    return jnp.transpose(out_t, (0, 2, 1, 3))


# speedup vs baseline: 16.7390x; 2.3911x over previous
import jax
import jax.numpy as jnp
from jax import lax
from jax.experimental import pallas as pl
from jax.experimental.pallas import tpu as pltpu, tpu_sc as plsc

MAX_NUM_TILES = 4
NUM_PATCHES = 1025
HIDDEN = 1280
BATCH = 8
NPH = NUM_PATCHES * HIDDEN
NSUB = 32


def _body(hid_t, tbl, tbl8, emb, rid, gat, out_t,
          tbuf, t8buf, ebuf, hb0, hb1, ob0, ob1,
          rid_v, gat_v,
          st, sh0, sh1, so0, so1):
    wid = lax.axis_index("s") * 2 + lax.axis_index("c")

    pltpu.sync_copy(rid, rid_v)
    pltpu.sync_copy(gat, gat_v)
    ridv = rid_v[...]
    graw = gat_v[...]
    ax = jnp.abs(graw)
    ex = jnp.exp(ax * 2.0)
    tpos = 1.0 - 2.0 / (ex + 1.0)
    gv = jnp.where(graw < 0.0, -tpos, tpos)
    omgv = 1.0 - gv
    lanes = lax.iota(jnp.int32, 16)
    any8 = jnp.max(ridv) == 8

    hb = (hb0, hb1)
    ob = (ob0, ob1)
    sh = (sh0, sh1)
    so = (so0, so1)

    n_k = jnp.where(wid == 0, 33, 32)

    def p_of(k):
        return wid + NSUB * k

    def start_tbl(k):
        p = p_of(k)
        c0 = p * HIDDEN
        for t in range(MAX_NUM_TILES):
            pltpu.async_copy(
                tbl.at[pl.ds(0, 8), pl.ds(t * NPH + c0, HIDDEN)],
                tbuf.at[t], st)
        pltpu.async_copy(emb.at[pl.ds((p // 8) * 8, 8)], ebuf, st)

        @pl.when(any8)
        def _():
            pltpu.async_copy(tbl8.at[p], t8buf, st)

    def wait_tbl():
        for t in range(MAX_NUM_TILES):
            pltpu.make_async_copy(
                tbl.at[pl.ds(0, 8), pl.ds(0, HIDDEN)], tbuf.at[t],
                st).wait()
        pltpu.make_async_copy(emb.at[pl.ds(0, 8)], ebuf, st).wait()

        @pl.when(any8)
        def _():
            pltpu.make_async_copy(tbl8.at[0], t8buf, st).wait()

    def start_h(k, b, s):
        pltpu.async_copy(hid_t.at[b, p_of(k)], hb[s], sh[s])

    def wait_h(s):
        pltpu.make_async_copy(hid_t.at[0, 0], hb[s], sh[s]).wait()

    def start_o(k, b, s):
        pltpu.async_copy(ob[s], out_t.at[b, p_of(k)], so[s])

    def wait_o(s):
        pltpu.make_async_copy(ob[s], out_t.at[0, 0], so[s]).wait()

    def compute(k, b, s_h):
        idb = jnp.sum(jnp.where(lanes == b, ridv, 0))
        pm8 = p_of(k) % 8
        hbuf, obuf = hb[s_h], ob[s_h]

        @pl.when(idb < 8)
        def _():
            for t in range(MAX_NUM_TILES):
                @plsc.parallel_loop(0, HIDDEN, step=16, unroll=8)
                def _(c, t=t):
                    obuf[t, pl.ds(c, 16)] = (
                        hbuf[t, pl.ds(c, 16)]
                        + omgv * ebuf[pm8, pl.ds(c, 16)]
                        + gv * tbuf[t, idb, pl.ds(c, 16)])

        @pl.when(idb == 8)
        def _():
            for t in range(MAX_NUM_TILES):
                @plsc.parallel_loop(0, HIDDEN, step=16, unroll=8)
                def _(c, t=t):
                    obuf[t, pl.ds(c, 16)] = (
                        hbuf[t, pl.ds(c, 16)]
                        + omgv * ebuf[pm8, pl.ds(c, 16)]
                        + gv * t8buf[t, pl.ds(c, 16)])

    start_tbl(0)
    start_h(0, 0, 0)

    def chunk(k, carry):
        wait_tbl()

        def bpair(q, carry2, k=k):
            for sb in (0, 1):
                b = 2 * q + sb
                wait_h(sb)
                if sb == 0:
                    start_h(k, b + 1, 1)
                else:
                    @pl.when(q < 3)
                    def _(k=k, b=b):
                        start_h(k, b + 1, 0)

                    @pl.when((q == 3) & (k + 1 < n_k))
                    def _(k=k):
                        start_h(k + 1, 0, 0)
                @pl.when((q >= 1) | (k > 0))
                def _(sb=sb):
                    wait_o(sb)
                compute(k, b, sb)
                start_o(k, b, sb)
            return carry2

        lax.fori_loop(0, BATCH // 2, bpair, 0)

        @pl.when(k + 1 < n_k)
        def _(k=k):
            start_tbl(k + 1)
        return carry

    lax.fori_loop(0, n_k, chunk, 0)
    wait_o(0)
    wait_o(1)


_PB = 25


def _stage8_body(t0_ref, t1_ref, t2_ref, t3_ref, o_ref):
    for t, t_ref in enumerate((t0_ref, t1_ref, t2_ref, t3_ref)):
        for r in range(_PB):
            o_ref[r, t, :] = t_ref[0, pl.ds(r * HIDDEN, HIDDEN)]


def _stage_row8(tile_table):
    return pl.pallas_call(
        _stage8_body,
        grid=(NUM_PATCHES // _PB,),
        in_specs=[pl.BlockSpec((8, _PB * HIDDEN),
                               lambda pb, t=t: (1, 41 * t + pb))
                  for t in range(MAX_NUM_TILES)],
        out_specs=pl.BlockSpec((_PB, MAX_NUM_TILES, HIDDEN),
                               lambda pb: (pb, 0, 0)),
        out_shape=jax.ShapeDtypeStruct(
            (NUM_PATCHES, MAX_NUM_TILES, HIDDEN), jnp.float32),
    )(tile_table, tile_table, tile_table, tile_table)


@jax.jit
def kernel(hidden_state, aspect_ratio_ids, gate, embedding, tile_table):
    hid_t = jnp.transpose(hidden_state, (0, 2, 1, 3))
    tbl8 = _stage_row8(tile_table)
    rid = jnp.zeros((16,), jnp.int32).at[:BATCH].set(
        aspect_ratio_ids.astype(jnp.int32))
    gat = jnp.broadcast_to(gate.astype(jnp.float32), (16,))

    run = pl.kernel(
        _body,
        out_type=jax.ShapeDtypeStruct(
            (BATCH, NUM_PATCHES, MAX_NUM_TILES, HIDDEN), jnp.float32),
        mesh=plsc.VectorSubcoreMesh(
            core_axis_name="c", subcore_axis_name="s",
            num_cores=2, num_subcores=16),
        compiler_params=pltpu.CompilerParams(needs_layout_passes=False),
        scratch_types=[
            pltpu.VMEM((MAX_NUM_TILES, 8, HIDDEN), jnp.float32),
            pltpu.VMEM((MAX_NUM_TILES, HIDDEN), jnp.float32),
            pltpu.VMEM((8, HIDDEN), jnp.float32),
            pltpu.VMEM((MAX_NUM_TILES, HIDDEN), jnp.float32),
            pltpu.VMEM((MAX_NUM_TILES, HIDDEN), jnp.float32),
            pltpu.VMEM((MAX_NUM_TILES, HIDDEN), jnp.float32),
            pltpu.VMEM((MAX_NUM_TILES, HIDDEN), jnp.float32),
            pltpu.VMEM((16,), jnp.int32),
            pltpu.VMEM((16,), jnp.float32),
            pltpu.SemaphoreType.DMA,
            pltpu.SemaphoreType.DMA,
            pltpu.SemaphoreType.DMA,
            pltpu.SemaphoreType.DMA,
            pltpu.SemaphoreType.DMA,
        ],
    )
    out_t = run(hid_t, tile_table, tbl8, embedding, rid, gat)
    return jnp.transpose(out_t, (0, 2, 1, 3))
